# baseline (speedup 1.0000x reference)
import functools

import jax
import jax.numpy as jnp
from jax import lax
from jax.experimental import pallas as pl
from jax.experimental.pallas import tpu as pltpu
from jax.experimental.pallas import tpu_sc as plsc

_CB = 1024
_D = 64
_BETA = 0.1
_BLOCK_T = 2304


def _argmin_body(x_ref, e_ref, idx_ref, loss_ref, e2_ref):
    nb = pl.num_programs(0)
    pid = pl.program_id(0)
    xb = x_ref[...]
    e = e_ref[...]

    @pl.when(pid == 0)
    def _():
        e2_ref[...] = jnp.sum(e * e, axis=1)[None, :]
        loss_ref[0, 0] = 0.0

    xe = lax.dot_general(xb, e, (((1,), (1,)), ((), ())),
                         preferred_element_type=jnp.float32)
    x2 = jnp.sum(xb * xb, axis=1, keepdims=True)
    dist = (x2 + e2_ref[...]) - 2.0 * xe
    minval = jnp.min(dist, axis=1, keepdims=True)
    iota_f = lax.broadcasted_iota(jnp.int32, dist.shape, 1).astype(jnp.float32)
    idx_f = jnp.min(jnp.where(dist == minval, iota_f, float(_CB)),
                    axis=1, keepdims=True)
    idx_ref[...] = idx_f.astype(jnp.int32)
    loss_ref[0, 0] += jnp.sum(minval)

    @pl.when(pid == nb - 1)
    def _():
        n = nb * _BLOCK_T * _D
        loss_ref[0, 0] *= (1.0 + _BETA) / n


def _gather_body(e_hbm, idx_hbm, out_hbm, idx_v, rows_v, sem):
    wid = lax.axis_index("s") * plsc.get_sparse_core_info().num_cores + lax.axis_index("c")
    bpw = idx_v.shape[0]
    base = wid * bpw
    pltpu.sync_copy(idx_hbm.at[pl.ds(base, bpw)], idx_v)
    pltpu.async_copy(e_hbm.at[idx_v], rows_v, sem).wait()
    pltpu.sync_copy(rows_v, out_hbm.at[pl.ds(base, bpw)])


@jax.jit
def _vq(x, embedding):
    b, t, d = x.shape
    nt = b * t
    x2d = x.reshape(nt, d)
    nb = nt // _BLOCK_T
    idx2, loss = pl.pallas_call(
        _argmin_body,
        grid=(nb,),
        in_specs=[
            pl.BlockSpec((_BLOCK_T, _D), lambda i: (i, 0)),
            pl.BlockSpec((_CB, _D), lambda i: (0, 0)),
        ],
        out_specs=[
            pl.BlockSpec((_BLOCK_T, 1), lambda i: (i, 0)),
            pl.BlockSpec((1, 1), lambda i: (0, 0),
                         memory_space=pltpu.SMEM),
        ],
        out_shape=[
            jax.ShapeDtypeStruct((nt, 1), jnp.int32),
            jax.ShapeDtypeStruct((1, 1), jnp.float32),
        ],
        scratch_shapes=[pltpu.VMEM((1, _CB), jnp.float32)],
    )(x2d, embedding)

    idx_flat = idx2.reshape(nt)
    sc = plsc.get_sparse_core_info()
    bpw = nt // (sc.num_cores * sc.num_subcores)
    e_pad = jnp.concatenate(
        [embedding, jnp.zeros((_CB, 128 - _D), jnp.float32)], axis=1)
    vals = pl.kernel(
        _gather_body,
        mesh=plsc.VectorSubcoreMesh(core_axis_name="c", subcore_axis_name="s"),
        out_type=jax.ShapeDtypeStruct((nt, 128), jnp.float32),
        scratch_types=[
            pltpu.VMEM((bpw,), jnp.int32),
            pltpu.VMEM((bpw, 128), jnp.float32),
            pltpu.SemaphoreType.DMA,
        ],
    )(e_pad, idx_flat)

    return (vals[:, :_D].reshape(b, t, d), idx_flat.reshape(b, t), loss[0, 0])


def kernel(x, embedding):
    return _vq(x, embedding)


# speedup vs baseline: 1.2862x; 1.2862x over previous
import functools

import jax
import jax.numpy as jnp
from jax import lax
from jax.experimental import pallas as pl
from jax.experimental.pallas import tpu as pltpu
from jax.experimental.pallas import tpu_sc as plsc

_CB = 1024
_D = 64
_BETA = 0.1
_BLOCK_T = 2304


def _argmin_body(x_ref, e_ref, idx_ref, vals_ref, loss_ref, e2_ref):
    nb = pl.num_programs(0)
    pid = pl.program_id(0)
    xb = x_ref[...]
    e = e_ref[...]

    @pl.when(pid == 0)
    def _():
        e2_ref[...] = jnp.sum(e * e, axis=1)[None, :]
        loss_ref[0, 0] = 0.0

    xe = lax.dot_general(xb, e, (((1,), (1,)), ((), ())),
                         preferred_element_type=jnp.float32)
    x2 = jnp.sum(xb * xb, axis=1, keepdims=True)
    dist = (x2 + e2_ref[...]) - 2.0 * xe
    minval = jnp.min(dist, axis=1, keepdims=True)
    iota_f = lax.broadcasted_iota(jnp.int32, dist.shape, 1).astype(jnp.float32)
    idx_f = jnp.min(jnp.where(dist == minval, iota_f, float(_CB)),
                    axis=1, keepdims=True)
    idx_ref[...] = idx_f.astype(jnp.int32)
    onehot = (iota_f == idx_f).astype(jnp.float32)
    vals_ref[...] = lax.dot_general(onehot, e, (((1,), (0,)), ((), ())),
                                    preferred_element_type=jnp.float32)
    loss_ref[0, 0] += jnp.sum(minval)

    @pl.when(pid == nb - 1)
    def _():
        n = nb * _BLOCK_T * _D
        loss_ref[0, 0] *= (1.0 + _BETA) / n


def _gather_body(e_hbm, idx_hbm, out_hbm, idx_v, rows_v, sem):
    wid = lax.axis_index("s") * plsc.get_sparse_core_info().num_cores + lax.axis_index("c")
    bpw = idx_v.shape[0]
    base = wid * bpw
    pltpu.sync_copy(idx_hbm.at[pl.ds(base, bpw)], idx_v)
    pltpu.async_copy(e_hbm.at[idx_v], rows_v, sem).wait()
    pltpu.sync_copy(rows_v, out_hbm.at[pl.ds(base, bpw)])


@jax.jit
def _vq(x, embedding):
    b, t, d = x.shape
    nt = b * t
    x2d = x.reshape(nt, d)
    nb = nt // _BLOCK_T
    idx2, vals, loss = pl.pallas_call(
        _argmin_body,
        grid=(nb,),
        in_specs=[
            pl.BlockSpec((_BLOCK_T, _D), lambda i: (i, 0)),
            pl.BlockSpec((_CB, _D), lambda i: (0, 0)),
        ],
        out_specs=[
            pl.BlockSpec((_BLOCK_T, 1), lambda i: (i, 0)),
            pl.BlockSpec((_BLOCK_T, _D), lambda i: (i, 0)),
            pl.BlockSpec((1, 1), lambda i: (0, 0),
                         memory_space=pltpu.SMEM),
        ],
        out_shape=[
            jax.ShapeDtypeStruct((nt, 1), jnp.int32),
            jax.ShapeDtypeStruct((nt, _D), jnp.float32),
            jax.ShapeDtypeStruct((1, 1), jnp.float32),
        ],
        scratch_shapes=[pltpu.VMEM((1, _CB), jnp.float32)],
    )(x2d, embedding)

    return (vals.reshape(b, t, d), idx2.reshape(b, t), loss[0, 0])


def kernel(x, embedding):
    return _vq(x, embedding)


# speedup vs baseline: 1.2864x; 1.0002x over previous
import jax
import jax.numpy as jnp
from jax import lax
from jax.experimental import pallas as pl
from jax.experimental.pallas import tpu as pltpu

_CB = 1024
_D = 64
_BETA = 0.1
_BLOCK_T = 2304


def _argmin_body(x_ref, e_ref, idx_ref, vals_ref, loss_ref, e2_ref):
    nb = pl.num_programs(0)
    pid = pl.program_id(0)
    xb = x_ref[...]
    e = e_ref[...]

    @pl.when(pid == 0)
    def _():
        e2_ref[...] = jnp.sum(e * e, axis=1)[None, :]
        loss_ref[0, 0] = 0.0

    xe = lax.dot_general(xb, e, (((1,), (1,)), ((), ())),
                         preferred_element_type=jnp.float32)
    x2 = jnp.sum(xb * xb, axis=1, keepdims=True)
    dist = (x2 + e2_ref[...]) - 2.0 * xe
    minval = jnp.min(dist, axis=1, keepdims=True)
    iota_f = lax.broadcasted_iota(jnp.int32, dist.shape, 1).astype(jnp.float32)
    idx_f = jnp.min(jnp.where(dist == minval, iota_f, float(_CB)),
                    axis=1, keepdims=True)
    idx_ref[...] = idx_f.astype(jnp.int32)
    onehot = (iota_f == idx_f).astype(jnp.float32)
    vals_ref[...] = lax.dot_general(onehot, e, (((1,), (0,)), ((), ())),
                                    preferred_element_type=jnp.float32)
    loss_ref[0, 0] += jnp.sum(minval)

    @pl.when(pid == nb - 1)
    def _():
        n = nb * _BLOCK_T * _D
        loss_ref[0, 0] *= (1.0 + _BETA) / n


@jax.jit
def _vq(x, embedding):
    b, t, d = x.shape
    nt = b * t
    x2d = x.reshape(nt, d)
    nb = nt // _BLOCK_T
    idx2, vals, loss = pl.pallas_call(
        _argmin_body,
        grid=(nb,),
        in_specs=[
            pl.BlockSpec((_BLOCK_T, _D), lambda i: (i, 0)),
            pl.BlockSpec((_CB, _D), lambda i: (0, 0)),
        ],
        out_specs=[
            pl.BlockSpec((_BLOCK_T, 1), lambda i: (i, 0)),
            pl.BlockSpec((_BLOCK_T, _D), lambda i: (i, 0)),
            pl.BlockSpec((1, 1), lambda i: (0, 0),
                         memory_space=pltpu.SMEM),
        ],
        out_shape=[
            jax.ShapeDtypeStruct((nt, 1), jnp.int32),
            jax.ShapeDtypeStruct((nt, _D), jnp.float32),
            jax.ShapeDtypeStruct((1, 1), jnp.float32),
        ],
        scratch_shapes=[pltpu.VMEM((1, _CB), jnp.float32)],
    )(x2d, embedding)

    return (vals.reshape(b, t, d), idx2.reshape(b, t), loss[0, 0])


def kernel(x, embedding):
    return _vq(x, embedding)
